# Initial kernel scaffold; baseline (speedup 1.0000x reference)
import jax
import jax.numpy as jnp
from jax.experimental import pallas as pl


def kernel(xyz, feats, W0, b0, g0, beta0, W1, b1, g1, beta1, W2, b2, g2, beta2):
    raise NotImplementedError("write your pallas kernel here")


# speedup vs baseline: 10.1104x; 10.1104x over previous
import functools

import jax
import jax.numpy as jnp
from jax import lax
from jax.experimental import pallas as pl
from jax.experimental.pallas import tpu as pltpu
from jax.experimental.pallas import tpu_sc as plsc


def _topk_body(K, P, c_ref, x_ref, knn_ref, cnt_ref):
    b = pl.program_id(0)
    cb = c_ref[0]
    xb = x_ref[0]
    cc = jnp.sum(cb * cb, axis=1, keepdims=True)
    xx = jnp.sum(xb * xb, axis=0, keepdims=True)
    cb16 = cb.astype(jnp.bfloat16).astype(jnp.float32)
    xb16 = xb.astype(jnp.bfloat16).astype(jnp.float32)
    cx = (cb16[:, 0:1] * xb16[0:1, :]
          + cb16[:, 1:2] * xb16[1:2, :]
          + cb16[:, 2:3] * xb16[2:3, :])
    d = cc + xx - 2.0 * cx
    iota = lax.broadcasted_iota(jnp.int32, d.shape, 1)
    inf = jnp.float32(jnp.inf)
    off = b * P
    for j in range(K):
        mn = jnp.min(d, axis=1, keepdims=True)
        am = jnp.min(jnp.where(d == mn, iota, jnp.int32(P)), axis=1,
                     keepdims=True)
        knn_ref[0, :, j:j + 1] = am + off
        d = jnp.where(iota == am, inf, d)
    sel = jnp.sum((d == inf).astype(jnp.float32), axis=0, keepdims=True)

    @pl.when(pl.program_id(1) == 0)
    def _init():
        cnt_ref[0] = sel

    @pl.when(pl.program_id(1) != 0)
    def _acc():
        cnt_ref[0] = cnt_ref[0] + sel


def _topk(centers, x_t, K, Mb, interpret=False):
    B, M, _ = centers.shape
    P = x_t.shape[2]
    nblk = M // Mb
    return pl.pallas_call(
        functools.partial(_topk_body, K, P),
        grid=(B, nblk),
        in_specs=[
            pl.BlockSpec((1, Mb, 3), lambda b, i: (b, i, 0)),
            pl.BlockSpec((1, 3, P), lambda b, i: (b, 0, 0)),
        ],
        out_specs=[
            pl.BlockSpec((1, Mb, K), lambda b, i: (b, i, 0)),
            pl.BlockSpec((1, 1, P), lambda b, i: (b, 0, 0)),
        ],
        out_shape=[
            jax.ShapeDtypeStruct((B, M, K), jnp.int32),
            jax.ShapeDtypeStruct((B, 1, P), jnp.float32),
        ],
        interpret=interpret,
    )(centers, x_t)


def _mlp_body(Ntot, pts_ref, cnt_ref,
              w0_ref, b0_ref, g0_ref, be0_ref,
              w1_ref, b1_ref, g1_ref, be1_ref,
              w2_ref, b2_ref, g2_ref, be2_ref,
              x3_ref):
    B = pts_ref.shape[0]
    cnt = cnt_ref[...]
    Ws = (w0_ref[...], w1_ref[...], w2_ref[...])
    bs = (b0_ref[...], b1_ref[...], b2_ref[...])
    gs = (g0_ref[...], g1_ref[...], g2_ref[...])
    bes = (be0_ref[...], be1_ref[...], be2_ref[...])

    def z_chain(b, L, affs):
        x = pts_ref[b]
        z = None
        for l in range(L + 1):
            z = jnp.dot(x, Ws[l], preferred_element_type=jnp.float32) + bs[l]
            if l < L:
                a, c = affs[l]
                x = jnp.maximum(z * a + c, 0.0)
        return z

    affs = []
    for L in range(3):
        oc = Ws[L].shape[1]
        s = jnp.zeros((1, oc), jnp.float32)
        q = jnp.zeros((1, oc), jnp.float32)
        for b in range(B):
            z = z_chain(b, L, affs)
            cb_ = cnt[b]
            s = s + jnp.dot(cb_, z, preferred_element_type=jnp.float32)
            q = q + jnp.dot(cb_, z * z, preferred_element_type=jnp.float32)
        mu = s / Ntot
        var = q / Ntot - mu * mu
        a = gs[L] / jnp.sqrt(var + 1e-5)
        c = bes[L] - mu * a
        affs.append((a, c))
    a2, c2 = affs[2]
    for b in range(B):
        z = z_chain(b, 2, affs)
        x3_ref[b] = jnp.maximum(z * a2 + c2, 0.0)


def _mlp(pts, cnt, params, Ntot, interpret=False):
    B, P, _ = pts.shape
    oc_last = params[8].shape[1]
    return pl.pallas_call(
        functools.partial(_mlp_body, Ntot),
        out_shape=jax.ShapeDtypeStruct((B, P, oc_last), jnp.float32),
        interpret=interpret,
    )(pts, cnt, *params)


def _gather_max(x3f, knn2, n_centers, K):
    D = x3f.shape[1]
    info = plsc.get_sparse_core_info()
    NW = info.num_cores * info.num_subcores
    TOT = knn2.shape[0]
    rows_w = TOT // NW
    cpw = n_centers // NW
    cpr = 128 // K
    nh = D // 16
    mesh = plsc.VectorSubcoreMesh(core_axis_name="c", subcore_axis_name="s")

    @functools.partial(
        pl.kernel, mesh=mesh,
        out_type=jax.ShapeDtypeStruct((n_centers, D), jnp.float32),
        scratch_types=[
            pltpu.VMEM((rows_w, 128), jnp.int32),
            pltpu.VMEM((128, D), jnp.float32),
            pltpu.VMEM((cpw, D), jnp.float32),
            pltpu.SemaphoreType.DMA,
        ],
    )
    def k(x3_hbm, idx_hbm, out_hbm, idx_v, rows_v, out_v, sem):
        wid = lax.axis_index("s") * info.num_cores + lax.axis_index("c")
        pltpu.sync_copy(idx_hbm.at[pl.ds(wid * rows_w, rows_w)], idx_v)

        def chunk_body(ch, _):
            pltpu.async_copy(x3_hbm.at[idx_v.at[ch]], rows_v, sem).wait()

            def center_body(ci, _):
                base = ci * K
                for h in range(nh):
                    sl = pl.ds(h * 16, 16)
                    acc = rows_v[base, sl]
                    for r in range(1, K):
                        acc = jnp.maximum(acc, rows_v[base + r, sl])
                    out_v[ch * cpr + ci, sl] = acc
                return 0

            lax.fori_loop(0, cpr, center_body, 0)
            return 0

        lax.fori_loop(0, rows_w, chunk_body, 0)
        pltpu.sync_copy(out_v, out_hbm.at[pl.ds(wid * cpw, cpw)])

    return k(x3f, knn2)


def kernel(xyz, feats, W0, b0, g0, beta0, W1, b1, g1, beta1,
           W2, b2, g2, beta2):
    B, P, _ = xyz.shape
    M = max(1, P // 4)
    K = min(32, P)
    idx_center = jnp.linspace(0.0, P - 1, M).astype(jnp.int32)
    centers = xyz[:, idx_center, :]
    x_t = jnp.transpose(xyz, (0, 2, 1))

    Mb = 256 if M % 256 == 0 else M
    knn, cnt = _topk(centers, x_t, K, Mb)

    pts = jnp.concatenate([xyz, jnp.transpose(feats, (0, 2, 1))], axis=2)
    params = (W0.T, b0.reshape(1, -1), g0.reshape(1, -1), beta0.reshape(1, -1),
              W1.T, b1.reshape(1, -1), g1.reshape(1, -1), beta1.reshape(1, -1),
              W2.T, b2.reshape(1, -1), g2.reshape(1, -1), beta2.reshape(1, -1))
    x3 = _mlp(pts, cnt, params, float(B * M * K))

    x3f = x3.reshape(B * P, x3.shape[2])
    knn2 = knn.reshape(B * M * K // 128, 128)
    outm = _gather_max(x3f, knn2, B * M, K)
    out = jnp.transpose(outm.reshape(B, M, x3.shape[2]), (0, 2, 1))
    return centers, out


# speedup vs baseline: 11.0474x; 1.0927x over previous
import functools

import jax
import jax.numpy as jnp
from jax import lax
from jax.experimental import pallas as pl
from jax.experimental.pallas import tpu as pltpu
from jax.experimental.pallas import tpu_sc as plsc


def _topk_body(K, P, c_ref, x_ref, knn_ref, cnt_ref):
    b = pl.program_id(0)
    cb = c_ref[0]
    xb = x_ref[0]
    cc = jnp.sum(cb * cb, axis=1, keepdims=True)
    xx = jnp.sum(xb * xb, axis=0, keepdims=True)
    cb16 = cb.astype(jnp.bfloat16).astype(jnp.float32)
    xb16 = xb.astype(jnp.bfloat16).astype(jnp.float32)
    cx = (cb16[:, 0:1] * xb16[0:1, :]
          + cb16[:, 1:2] * xb16[1:2, :]
          + cb16[:, 2:3] * xb16[2:3, :])
    d = cc + xx - 2.0 * cx
    iota = lax.broadcasted_iota(jnp.int32, d.shape, 1)
    inf = jnp.float32(jnp.inf)
    off = b * P
    for j in range(K):
        mn = jnp.min(d, axis=1, keepdims=True)
        am = jnp.min(jnp.where(d == mn, iota, jnp.int32(P)), axis=1,
                     keepdims=True)
        knn_ref[0, :, j:j + 1] = am + off
        d = jnp.where(iota == am, inf, d)
    sel = jnp.sum((d == inf).astype(jnp.float32), axis=0, keepdims=True)

    @pl.when(pl.program_id(1) == 0)
    def _init():
        cnt_ref[0] = sel

    @pl.when(pl.program_id(1) != 0)
    def _acc():
        cnt_ref[0] = cnt_ref[0] + sel


def _topk(centers, x_t, K, Mb, interpret=False):
    B, M, _ = centers.shape
    P = x_t.shape[2]
    nblk = M // Mb
    return pl.pallas_call(
        functools.partial(_topk_body, K, P),
        grid=(B, nblk),
        in_specs=[
            pl.BlockSpec((1, Mb, 3), lambda b, i: (b, i, 0)),
            pl.BlockSpec((1, 3, P), lambda b, i: (b, 0, 0)),
        ],
        out_specs=[
            pl.BlockSpec((1, Mb, K), lambda b, i: (b, i, 0)),
            pl.BlockSpec((1, 1, P), lambda b, i: (b, 0, 0)),
        ],
        out_shape=[
            jax.ShapeDtypeStruct((B, M, K), jnp.int32),
            jax.ShapeDtypeStruct((B, 1, P), jnp.float32),
        ],
        interpret=interpret,
    )(centers, x_t)


def _mlp_body(Ntot, pts_ref, cnt_ref,
              w0_ref, b0_ref, g0_ref, be0_ref,
              w1_ref, b1_ref, g1_ref, be1_ref,
              w2_ref, b2_ref, g2_ref, be2_ref,
              x3_ref):
    B = pts_ref.shape[0]
    cnt = cnt_ref[...]
    Ws = (w0_ref[...], w1_ref[...], w2_ref[...])
    bs = (b0_ref[...], b1_ref[...], b2_ref[...])
    gs = (g0_ref[...], g1_ref[...], g2_ref[...])
    bes = (be0_ref[...], be1_ref[...], be2_ref[...])

    def z_chain(b, L, affs):
        x = pts_ref[b]
        z = None
        for l in range(L + 1):
            z = jnp.dot(x, Ws[l], preferred_element_type=jnp.float32) + bs[l]
            if l < L:
                a, c = affs[l]
                x = jnp.maximum(z * a + c, 0.0)
        return z

    affs = []
    for L in range(3):
        oc = Ws[L].shape[1]
        s = jnp.zeros((1, oc), jnp.float32)
        q = jnp.zeros((1, oc), jnp.float32)
        for b in range(B):
            z = z_chain(b, L, affs)
            cb_ = cnt[b]
            s = s + jnp.dot(cb_, z, preferred_element_type=jnp.float32)
            q = q + jnp.dot(cb_, z * z, preferred_element_type=jnp.float32)
        mu = s / Ntot
        var = q / Ntot - mu * mu
        a = gs[L] / jnp.sqrt(var + 1e-5)
        c = bes[L] - mu * a
        affs.append((a, c))
    a2, c2 = affs[2]
    for b in range(B):
        z = z_chain(b, 2, affs)
        x3_ref[b] = jnp.maximum(z * a2 + c2, 0.0)


def _mlp(pts, cnt, params, Ntot, interpret=False):
    B, P, _ = pts.shape
    oc_last = params[8].shape[1]
    return pl.pallas_call(
        functools.partial(_mlp_body, Ntot),
        out_shape=jax.ShapeDtypeStruct((B, P, oc_last), jnp.float32),
        interpret=interpret,
    )(pts, cnt, *params)


def _gather_max(x3f, knn2, n_centers, K):
    D = x3f.shape[1]
    info = plsc.get_sparse_core_info()
    NW = info.num_cores * info.num_subcores
    TOT = knn2.shape[0]
    rows_w = TOT // NW
    cpw = n_centers // NW
    cpr = 128 // K
    nh = D // 16
    mesh = plsc.VectorSubcoreMesh(core_axis_name="c", subcore_axis_name="s")

    @functools.partial(
        pl.kernel, mesh=mesh,
        out_type=jax.ShapeDtypeStruct((n_centers, D), jnp.float32),
        scratch_types=[
            pltpu.VMEM((rows_w, 128), jnp.int32),
            pltpu.VMEM((128, D), jnp.float32),
            pltpu.VMEM((cpw, D), jnp.float32),
            pltpu.SemaphoreType.DMA,
        ],
    )
    def k(x3_hbm, idx_hbm, out_hbm, idx_v, rows_v, out_v, sem):
        wid = lax.axis_index("s") * info.num_cores + lax.axis_index("c")
        pltpu.sync_copy(idx_hbm.at[pl.ds(wid * rows_w, rows_w)], idx_v)

        def chunk_body(ch, _):
            pltpu.async_copy(x3_hbm.at[idx_v.at[ch]], rows_v, sem).wait()

            def center_body(ci, _):
                base = ci * K
                for h in range(nh):
                    sl = pl.ds(h * 16, 16)
                    acc = rows_v[base, sl]
                    for r in range(1, K):
                        acc = jnp.maximum(acc, rows_v[base + r, sl])
                    out_v[ch * cpr + ci, sl] = acc
                return 0

            lax.fori_loop(0, cpr, center_body, 0)
            return 0

        lax.fori_loop(0, rows_w, chunk_body, 0)
        pltpu.sync_copy(out_v, out_hbm.at[pl.ds(wid * cpw, cpw)])

    return k(x3f, knn2)


def kernel(xyz, feats, W0, b0, g0, beta0, W1, b1, g1, beta1,
           W2, b2, g2, beta2):
    B, P, _ = xyz.shape
    M = max(1, P // 4)
    K = min(32, P)
    idx_center = jnp.linspace(0.0, P - 1, M).astype(jnp.int32)
    centers = xyz[:, idx_center, :]
    x_t = jnp.transpose(xyz, (0, 2, 1))

    Mb = 512 if M % 512 == 0 else M
    knn, cnt = _topk(centers, x_t, K, Mb)

    pts = jnp.concatenate([xyz, jnp.transpose(feats, (0, 2, 1))], axis=2)
    params = (W0.T, b0.reshape(1, -1), g0.reshape(1, -1), beta0.reshape(1, -1),
              W1.T, b1.reshape(1, -1), g1.reshape(1, -1), beta1.reshape(1, -1),
              W2.T, b2.reshape(1, -1), g2.reshape(1, -1), beta2.reshape(1, -1))
    x3 = _mlp(pts, cnt, params, float(B * M * K))

    x3f = x3.reshape(B * P, x3.shape[2])
    knn2 = knn.reshape(B * M * K // 128, 128)
    outm = _gather_max(x3f, knn2, B * M, K)
    out = jnp.transpose(outm.reshape(B, M, x3.shape[2]), (0, 2, 1))
    return centers, out


# speedup vs baseline: 11.7035x; 1.0594x over previous
import functools

import jax
import jax.numpy as jnp
from jax import lax
from jax.experimental import pallas as pl
from jax.experimental.pallas import tpu as pltpu
from jax.experimental.pallas import tpu_sc as plsc


def _topk_body(K, P, c_ref, x_ref, knn_ref, cnt_ref):
    b = pl.program_id(0)
    cb = c_ref[0]
    xb = x_ref[0]
    cc = jnp.sum(cb * cb, axis=1, keepdims=True)
    xx = jnp.sum(xb * xb, axis=0, keepdims=True)
    cb16 = cb.astype(jnp.bfloat16).astype(jnp.float32)
    xb16 = xb.astype(jnp.bfloat16).astype(jnp.float32)
    cx = (cb16[:, 0:1] * xb16[0:1, :]
          + cb16[:, 1:2] * xb16[1:2, :]
          + cb16[:, 2:3] * xb16[2:3, :])
    d = cc + xx - 2.0 * cx
    iota = lax.broadcasted_iota(jnp.int32, d.shape, 1)
    inf = jnp.float32(jnp.inf)
    off = b * P
    for j in range(K):
        am = jnp.argmin(d, axis=1)[:, None]
        knn_ref[0, :, j:j + 1] = am + off
        d = jnp.where(iota == am, inf, d)
    sel = jnp.sum((d == inf).astype(jnp.float32), axis=0, keepdims=True)

    @pl.when(pl.program_id(1) == 0)
    def _init():
        cnt_ref[0] = sel

    @pl.when(pl.program_id(1) != 0)
    def _acc():
        cnt_ref[0] = cnt_ref[0] + sel


def _topk(centers, x_t, K, Mb, interpret=False):
    B, M, _ = centers.shape
    P = x_t.shape[2]
    nblk = M // Mb
    return pl.pallas_call(
        functools.partial(_topk_body, K, P),
        grid=(B, nblk),
        in_specs=[
            pl.BlockSpec((1, Mb, 3), lambda b, i: (b, i, 0)),
            pl.BlockSpec((1, 3, P), lambda b, i: (b, 0, 0)),
        ],
        out_specs=[
            pl.BlockSpec((1, Mb, K), lambda b, i: (b, i, 0)),
            pl.BlockSpec((1, 1, P), lambda b, i: (b, 0, 0)),
        ],
        out_shape=[
            jax.ShapeDtypeStruct((B, M, K), jnp.int32),
            jax.ShapeDtypeStruct((B, 1, P), jnp.float32),
        ],
        interpret=interpret,
    )(centers, x_t)


def _mlp_body(Ntot, pts_ref, cnt_ref,
              w0_ref, b0_ref, g0_ref, be0_ref,
              w1_ref, b1_ref, g1_ref, be1_ref,
              w2_ref, b2_ref, g2_ref, be2_ref,
              x3_ref):
    B = pts_ref.shape[0]
    cnt = cnt_ref[...]
    Ws = (w0_ref[...], w1_ref[...], w2_ref[...])
    bs = (b0_ref[...], b1_ref[...], b2_ref[...])
    gs = (g0_ref[...], g1_ref[...], g2_ref[...])
    bes = (be0_ref[...], be1_ref[...], be2_ref[...])

    def z_chain(b, L, affs):
        x = pts_ref[b]
        z = None
        for l in range(L + 1):
            z = jnp.dot(x, Ws[l], preferred_element_type=jnp.float32) + bs[l]
            if l < L:
                a, c = affs[l]
                x = jnp.maximum(z * a + c, 0.0)
        return z

    affs = []
    for L in range(3):
        oc = Ws[L].shape[1]
        s = jnp.zeros((1, oc), jnp.float32)
        q = jnp.zeros((1, oc), jnp.float32)
        for b in range(B):
            z = z_chain(b, L, affs)
            cb_ = cnt[b]
            s = s + jnp.dot(cb_, z, preferred_element_type=jnp.float32)
            q = q + jnp.dot(cb_, z * z, preferred_element_type=jnp.float32)
        mu = s / Ntot
        var = q / Ntot - mu * mu
        a = gs[L] / jnp.sqrt(var + 1e-5)
        c = bes[L] - mu * a
        affs.append((a, c))
    a2, c2 = affs[2]
    for b in range(B):
        z = z_chain(b, 2, affs)
        x3_ref[b] = jnp.maximum(z * a2 + c2, 0.0)


def _mlp(pts, cnt, params, Ntot, interpret=False):
    B, P, _ = pts.shape
    oc_last = params[8].shape[1]
    return pl.pallas_call(
        functools.partial(_mlp_body, Ntot),
        out_shape=jax.ShapeDtypeStruct((B, P, oc_last), jnp.float32),
        interpret=interpret,
    )(pts, cnt, *params)


def _gather_max(x3f, knn2, n_centers, K):
    D = x3f.shape[1]
    info = plsc.get_sparse_core_info()
    NW = info.num_cores * info.num_subcores
    TOT = knn2.shape[0]
    rows_w = TOT // NW
    cpw = n_centers // NW
    cpr = 128 // K
    nh = D // 16
    mesh = plsc.VectorSubcoreMesh(core_axis_name="c", subcore_axis_name="s")

    @functools.partial(
        pl.kernel, mesh=mesh,
        out_type=jax.ShapeDtypeStruct((n_centers, D), jnp.float32),
        scratch_types=[
            pltpu.VMEM((rows_w, 128), jnp.int32),
            pltpu.VMEM((128, D), jnp.float32),
            pltpu.VMEM((cpw, D), jnp.float32),
            pltpu.SemaphoreType.DMA,
        ],
    )
    def k(x3_hbm, idx_hbm, out_hbm, idx_v, rows_v, out_v, sem):
        wid = lax.axis_index("s") * info.num_cores + lax.axis_index("c")
        pltpu.sync_copy(idx_hbm.at[pl.ds(wid * rows_w, rows_w)], idx_v)

        def chunk_body(ch, _):
            pltpu.async_copy(x3_hbm.at[idx_v.at[ch]], rows_v, sem).wait()

            def center_body(ci, _):
                base = ci * K
                for h in range(nh):
                    sl = pl.ds(h * 16, 16)
                    acc = rows_v[base, sl]
                    for r in range(1, K):
                        acc = jnp.maximum(acc, rows_v[base + r, sl])
                    out_v[ch * cpr + ci, sl] = acc
                return 0

            lax.fori_loop(0, cpr, center_body, 0)
            return 0

        lax.fori_loop(0, rows_w, chunk_body, 0)
        pltpu.sync_copy(out_v, out_hbm.at[pl.ds(wid * cpw, cpw)])

    return k(x3f, knn2)


def kernel(xyz, feats, W0, b0, g0, beta0, W1, b1, g1, beta1,
           W2, b2, g2, beta2):
    B, P, _ = xyz.shape
    M = max(1, P // 4)
    K = min(32, P)
    idx_center = jnp.linspace(0.0, P - 1, M).astype(jnp.int32)
    centers = xyz[:, idx_center, :]
    x_t = jnp.transpose(xyz, (0, 2, 1))

    Mb = 512 if M % 512 == 0 else M
    knn, cnt = _topk(centers, x_t, K, Mb)

    pts = jnp.concatenate([xyz, jnp.transpose(feats, (0, 2, 1))], axis=2)
    params = (W0.T, b0.reshape(1, -1), g0.reshape(1, -1), beta0.reshape(1, -1),
              W1.T, b1.reshape(1, -1), g1.reshape(1, -1), beta1.reshape(1, -1),
              W2.T, b2.reshape(1, -1), g2.reshape(1, -1), beta2.reshape(1, -1))
    x3 = _mlp(pts, cnt, params, float(B * M * K))

    x3f = x3.reshape(B * P, x3.shape[2])
    knn2 = knn.reshape(B * M * K // 128, 128)
    outm = _gather_max(x3f, knn2, B * M, K)
    out = jnp.transpose(outm.reshape(B, M, x3.shape[2]), (0, 2, 1))
    return centers, out


# speedup vs baseline: 11.7189x; 1.0013x over previous
import functools

import jax
import jax.numpy as jnp
from jax import lax
from jax.experimental import pallas as pl
from jax.experimental.pallas import tpu as pltpu
from jax.experimental.pallas import tpu_sc as plsc


def _topk_body(K, P, c_ref, x_ref, knn_ref, cnt_ref):
    b = pl.program_id(0)
    cb = c_ref[0]
    xb = x_ref[0]
    cc = jnp.sum(cb * cb, axis=1, keepdims=True)
    xx = jnp.sum(xb * xb, axis=0, keepdims=True)
    cb16 = cb.astype(jnp.bfloat16).astype(jnp.float32)
    xb16 = xb.astype(jnp.bfloat16).astype(jnp.float32)
    cx = (cb16[:, 0:1] * xb16[0:1, :]
          + cb16[:, 1:2] * xb16[1:2, :]
          + cb16[:, 2:3] * xb16[2:3, :])
    d = cc + xx - 2.0 * cx
    iota = lax.broadcasted_iota(jnp.int32, d.shape, 1)
    inf = jnp.float32(jnp.inf)
    off = b * P
    ams = []
    for j in range(K):
        am = jnp.argmin(d, axis=1)[:, None]
        ams.append(am)
        d = jnp.where(iota == am, inf, d)
    knn_ref[0] = jnp.concatenate(ams, axis=1) + off
    sel = jnp.sum((d == inf).astype(jnp.float32), axis=0, keepdims=True)

    @pl.when(pl.program_id(1) == 0)
    def _init():
        cnt_ref[0] = sel

    @pl.when(pl.program_id(1) != 0)
    def _acc():
        cnt_ref[0] = cnt_ref[0] + sel


def _topk(centers, x_t, K, Mb, interpret=False):
    B, M, _ = centers.shape
    P = x_t.shape[2]
    nblk = M // Mb
    return pl.pallas_call(
        functools.partial(_topk_body, K, P),
        grid=(B, nblk),
        in_specs=[
            pl.BlockSpec((1, Mb, 3), lambda b, i: (b, i, 0)),
            pl.BlockSpec((1, 3, P), lambda b, i: (b, 0, 0)),
        ],
        out_specs=[
            pl.BlockSpec((1, Mb, K), lambda b, i: (b, i, 0)),
            pl.BlockSpec((1, 1, P), lambda b, i: (b, 0, 0)),
        ],
        out_shape=[
            jax.ShapeDtypeStruct((B, M, K), jnp.int32),
            jax.ShapeDtypeStruct((B, 1, P), jnp.float32),
        ],
        interpret=interpret,
    )(centers, x_t)


def _mlp_body(Ntot, pts_ref, cnt_ref,
              w0_ref, b0_ref, g0_ref, be0_ref,
              w1_ref, b1_ref, g1_ref, be1_ref,
              w2_ref, b2_ref, g2_ref, be2_ref,
              x3_ref):
    B = pts_ref.shape[0]
    cnt = cnt_ref[...]
    Ws = (w0_ref[...], w1_ref[...], w2_ref[...])
    bs = (b0_ref[...], b1_ref[...], b2_ref[...])
    gs = (g0_ref[...], g1_ref[...], g2_ref[...])
    bes = (be0_ref[...], be1_ref[...], be2_ref[...])

    def z_chain(b, L, affs):
        x = pts_ref[b]
        z = None
        for l in range(L + 1):
            z = jnp.dot(x, Ws[l], preferred_element_type=jnp.float32) + bs[l]
            if l < L:
                a, c = affs[l]
                x = jnp.maximum(z * a + c, 0.0)
        return z

    affs = []
    for L in range(3):
        oc = Ws[L].shape[1]
        s = jnp.zeros((1, oc), jnp.float32)
        q = jnp.zeros((1, oc), jnp.float32)
        for b in range(B):
            z = z_chain(b, L, affs)
            cb_ = cnt[b]
            s = s + jnp.dot(cb_, z, preferred_element_type=jnp.float32)
            q = q + jnp.dot(cb_, z * z, preferred_element_type=jnp.float32)
        mu = s / Ntot
        var = q / Ntot - mu * mu
        a = gs[L] / jnp.sqrt(var + 1e-5)
        c = bes[L] - mu * a
        affs.append((a, c))
    a2, c2 = affs[2]
    for b in range(B):
        z = z_chain(b, 2, affs)
        x3_ref[b] = jnp.maximum(z * a2 + c2, 0.0)


def _mlp(pts, cnt, params, Ntot, interpret=False):
    B, P, _ = pts.shape
    oc_last = params[8].shape[1]
    return pl.pallas_call(
        functools.partial(_mlp_body, Ntot),
        out_shape=jax.ShapeDtypeStruct((B, P, oc_last), jnp.float32),
        interpret=interpret,
    )(pts, cnt, *params)


def _gather_max(x3f, knn2, n_centers, K):
    D = x3f.shape[1]
    info = plsc.get_sparse_core_info()
    NW = info.num_cores * info.num_subcores
    TOT = knn2.shape[0]
    rows_w = TOT // NW
    cpw = n_centers // NW
    cpr = 128 // K
    nh = D // 16
    mesh = plsc.VectorSubcoreMesh(core_axis_name="c", subcore_axis_name="s")

    @functools.partial(
        pl.kernel, mesh=mesh,
        out_type=jax.ShapeDtypeStruct((n_centers, D), jnp.float32),
        scratch_types=[
            pltpu.VMEM((rows_w, 128), jnp.int32),
            pltpu.VMEM((128, D), jnp.float32),
            pltpu.VMEM((cpw, D), jnp.float32),
            pltpu.SemaphoreType.DMA,
        ],
    )
    def k(x3_hbm, idx_hbm, out_hbm, idx_v, rows_v, out_v, sem):
        wid = lax.axis_index("s") * info.num_cores + lax.axis_index("c")
        pltpu.sync_copy(idx_hbm.at[pl.ds(wid * rows_w, rows_w)], idx_v)

        def chunk_body(ch, _):
            pltpu.async_copy(x3_hbm.at[idx_v.at[ch]], rows_v, sem).wait()

            def center_body(ci, _):
                base = ci * K
                for h in range(nh):
                    sl = pl.ds(h * 16, 16)
                    acc = rows_v[base, sl]
                    for r in range(1, K):
                        acc = jnp.maximum(acc, rows_v[base + r, sl])
                    out_v[ch * cpr + ci, sl] = acc
                return 0

            lax.fori_loop(0, cpr, center_body, 0)
            return 0

        lax.fori_loop(0, rows_w, chunk_body, 0)
        pltpu.sync_copy(out_v, out_hbm.at[pl.ds(wid * cpw, cpw)])

    return k(x3f, knn2)


def kernel(xyz, feats, W0, b0, g0, beta0, W1, b1, g1, beta1,
           W2, b2, g2, beta2):
    B, P, _ = xyz.shape
    M = max(1, P // 4)
    K = min(32, P)
    idx_center = jnp.linspace(0.0, P - 1, M).astype(jnp.int32)
    centers = xyz[:, idx_center, :]
    x_t = jnp.transpose(xyz, (0, 2, 1))

    Mb = 512 if M % 512 == 0 else M
    knn, cnt = _topk(centers, x_t, K, Mb)

    pts = jnp.concatenate([xyz, jnp.transpose(feats, (0, 2, 1))], axis=2)
    params = (W0.T, b0.reshape(1, -1), g0.reshape(1, -1), beta0.reshape(1, -1),
              W1.T, b1.reshape(1, -1), g1.reshape(1, -1), beta1.reshape(1, -1),
              W2.T, b2.reshape(1, -1), g2.reshape(1, -1), beta2.reshape(1, -1))
    x3 = _mlp(pts, cnt, params, float(B * M * K))

    x3f = x3.reshape(B * P, x3.shape[2])
    knn2 = knn.reshape(B * M * K // 128, 128)
    outm = _gather_max(x3f, knn2, B * M, K)
    out = jnp.transpose(outm.reshape(B, M, x3.shape[2]), (0, 2, 1))
    return centers, out
